# baseline (speedup 1.0000x reference)
import jax
import jax.numpy as jnp
from jax import lax
from jax.experimental import pallas as pl
from jax.experimental.pallas import tpu as pltpu
from jax.experimental.pallas import tpu_sc as plsc

DIM = 64

NC = 2
NS = 16
NW = NC * NS

IDXW = 128
GPC = 4
CHUNK = IDXW * GPC


def _gather_body(idx_hbm, table_hbm, out_hbm, idx_v, rows0, rows1, sem0, sem1):
    wid = lax.axis_index("s") * NC + lax.axis_index("c")
    rows_per_w = idx_hbm.shape[1] * idx_hbm.shape[2]
    n_chunks = rows_per_w // CHUNK
    base = wid * rows_per_w

    pltpu.sync_copy(idx_hbm.at[wid], idx_v)

    bufs = (rows0, rows1)
    sems = (sem0, sem1)

    def fire(c, buf, sem):
        for j in range(GPC):
            pltpu.async_copy(
                table_hbm.at[idx_v.at[c * GPC + j]],
                buf.at[pl.ds(j * IDXW, IDXW)],
                sem,
            )

    def drain(buf, sem):
        pltpu.make_async_copy(table_hbm.at[pl.ds(0, CHUNK)], buf, sem).wait()

    fire(0, rows0, sem0)
    fire(1, rows1, sem1)

    def body(i, carry):
        for b in range(2):
            c = i * 2 + b
            drain(bufs[b], sems[b])
            pltpu.sync_copy(bufs[b], out_hbm.at[pl.ds(base + c * CHUNK, CHUNK)])
            fire(c + 2, bufs[b], sems[b])
        return carry

    lax.fori_loop(0, (n_chunks - 2) // 2, body, 0)

    for b in range(2):
        c = n_chunks - 2 + b
        drain(bufs[b], sems[b])
        pltpu.sync_copy(bufs[b], out_hbm.at[pl.ds(base + c * CHUNK, CHUNK)])


def kernel(x, table):
    batch, fields = x.shape
    total = batch * fields
    rows_per_w = total // NW
    idx3 = x.reshape(NW, rows_per_w // IDXW, IDXW)

    out = pl.kernel(
        _gather_body,
        out_type=jax.ShapeDtypeStruct((total, DIM), jnp.float32),
        mesh=plsc.VectorSubcoreMesh(core_axis_name="c", subcore_axis_name="s"),
        compiler_params=pltpu.CompilerParams(use_tc_tiling_on_sc=False),
        scratch_types=[
            pltpu.VMEM((rows_per_w // IDXW, IDXW), jnp.int32),
            pltpu.VMEM((CHUNK, DIM), jnp.float32),
            pltpu.VMEM((CHUNK, DIM), jnp.float32),
            pltpu.SemaphoreType.DMA,
            pltpu.SemaphoreType.DMA,
        ],
    )(idx3, table)
    return out.reshape(batch, fields, DIM)


# speedup vs baseline: 1.0243x; 1.0243x over previous
import jax
import jax.numpy as jnp
from jax import lax
from jax.experimental import pallas as pl
from jax.experimental.pallas import tpu as pltpu
from jax.experimental.pallas import tpu_sc as plsc

DIM = 64

NC = 2
NS = 16
NW = NC * NS

IDXW = 128
GPC = 2
CHUNK = IDXW * GPC
WIDE = 128


def _gather_body(idx_hbm, table_hbm, out_hbm, idx_v, rows0, rows1, sem0, sem1):
    wid = lax.axis_index("s") * NC + lax.axis_index("c")
    rows_per_w = idx_hbm.shape[1] * idx_hbm.shape[2]
    n_chunks = rows_per_w // CHUNK
    base = wid * rows_per_w

    pltpu.sync_copy(idx_hbm.at[wid], idx_v)

    bufs = (rows0, rows1)
    sems = (sem0, sem1)

    def fire(c, buf, sem):
        for j in range(GPC):
            pltpu.async_copy(
                table_hbm.at[idx_v.at[c * GPC + j]],
                buf.at[pl.ds(j * IDXW, IDXW)],
                sem,
            )

    def drain(buf, sem):
        pltpu.make_async_copy(table_hbm.at[pl.ds(0, CHUNK)], buf, sem).wait()

    fire(0, rows0, sem0)
    fire(1, rows1, sem1)

    def body(i, carry):
        for b in range(2):
            c = i * 2 + b
            drain(bufs[b], sems[b])
            pltpu.sync_copy(bufs[b].at[:, pl.ds(0, DIM)],
                            out_hbm.at[pl.ds(base + c * CHUNK, CHUNK)])
            fire(c + 2, bufs[b], sems[b])
        return carry

    lax.fori_loop(0, (n_chunks - 2) // 2, body, 0)

    for b in range(2):
        c = n_chunks - 2 + b
        drain(bufs[b], sems[b])
        pltpu.sync_copy(bufs[b].at[:, pl.ds(0, DIM)],
                        out_hbm.at[pl.ds(base + c * CHUNK, CHUNK)])


def kernel(x, table):
    batch, fields = x.shape
    total = batch * fields
    rows_per_w = total // NW
    idx3 = x.reshape(NW, rows_per_w // IDXW, IDXW)
    table_wide = jnp.pad(table, ((0, 0), (0, 128 - DIM)))

    out = pl.kernel(
        _gather_body,
        out_type=jax.ShapeDtypeStruct((total, DIM), jnp.float32),
        mesh=plsc.VectorSubcoreMesh(core_axis_name="c", subcore_axis_name="s"),
        compiler_params=pltpu.CompilerParams(use_tc_tiling_on_sc=False),
        scratch_types=[
            pltpu.VMEM((rows_per_w // IDXW, IDXW), jnp.int32),
            pltpu.VMEM((CHUNK, WIDE), jnp.float32),
            pltpu.VMEM((CHUNK, WIDE), jnp.float32),
            pltpu.SemaphoreType.DMA,
            pltpu.SemaphoreType.DMA,
        ],
    )(idx3, table_wide)
    return out.reshape(batch, fields, DIM)
